# Initial kernel scaffold; baseline (speedup 1.0000x reference)
import jax
import jax.numpy as jnp
from jax.experimental import pallas as pl


def kernel(x, edge_index, batch, edge_attr, We0, be0, eps0, m0W1, m0b1, m0g, m0be, m0W2, m0b2, g0, bb0, We1, be1, eps1, m1W1, m1b1, m1g, m1be, m1W2, m1b2, g1, bb1, Wl, bl):
    raise NotImplementedError("write your pallas kernel here")


# speedup vs baseline: 5.6726x; 5.6726x over previous
import functools

import jax
import jax.numpy as jnp
from jax import lax
from jax.experimental import pallas as pl
from jax.experimental.pallas import tpu as pltpu
from jax.experimental.pallas import tpu_sc as plsc

N = 10000
E = 320000
D = 128
G = 64
OUT = 10

NC = 2
NS = 16
NW = NC * NS
EW = E // NW
C = 80
NCHUNK = EW // C
NGRP = C // 16
ROWS_PER_TILE = N // NS


def _lane_bcast(v, k):
  idx = jnp.full((16,), k, dtype=jnp.int32)
  dn = lax.GatherDimensionNumbers(
      offset_dims=(), collapsed_slice_dims=(0,), start_index_map=(0,))
  return lax.gather(v, idx[:, None], dn, (1,),
                    mode=lax.GatherScatterMode.PROMISE_IN_BOUNDS)


def _edge_body(xp, src_h, dst_h, ea_h, we_h, zeros_h, out,
               src_v, dst_v, ea_v, rows_v, we_v, out_acc, sem):
  cid = lax.axis_index("c")
  sid = lax.axis_index("s")
  wid = sid * NC + cid

  zsl = pl.ds(sid * ROWS_PER_TILE, ROWS_PER_TILE)
  pltpu.sync_copy(zeros_h.at[zsl], out_acc.at[zsl])
  plsc.subcore_barrier()

  pltpu.sync_copy(src_h.at[wid], src_v)
  pltpu.sync_copy(dst_h.at[wid], dst_v)
  pltpu.sync_copy(ea_h.at[wid], ea_v)
  pltpu.sync_copy(we_h, we_v)

  we_g = [we_v[pl.ds(f * 16, 16)] for f in range(8)]

  def chunk(c, _):
    pltpu.async_copy(xp.at[src_v.at[c]], rows_v, sem).wait()
    def grp(g, _):
      ea_grp = ea_v[c, pl.ds(g * 16, 16)]
      for k in range(16):
        ea_b = _lane_bcast(ea_grp, k)
        r = g * 16 + k
        for f in range(8):
          sl = pl.ds(f * 16, 16)
          rows_v[r, sl] = jnp.maximum(rows_v[r, sl] + ea_b * we_g[f], 0.0)
      return 0
    lax.fori_loop(0, NGRP, grp, 0)
    pltpu.sync_copy(rows_v, out_acc.at[dst_v.at[c]], add=True)
    return 0

  lax.fori_loop(0, NCHUNK, chunk, 0)
  plsc.subcore_barrier()
  pltpu.sync_copy(out_acc.at[zsl], out.at[cid, zsl])


def _make_edge_kernel():
  mesh = plsc.VectorSubcoreMesh(core_axis_name="c", subcore_axis_name="s")
  return pl.kernel(
      _edge_body,
      out_type=jax.ShapeDtypeStruct((NC, N, D), jnp.float32),
      mesh=mesh,
      compiler_params=pltpu.CompilerParams(use_tc_tiling_on_sc=False),
      scratch_types=[
          pltpu.VMEM((NCHUNK, C), jnp.int32),
          pltpu.VMEM((NCHUNK, C), jnp.int32),
          pltpu.VMEM((NCHUNK, C), jnp.float32),
          pltpu.VMEM((C, D), jnp.float32),
          pltpu.VMEM((D,), jnp.float32),
          pltpu.VMEM_SHARED((N, D), jnp.float32),
          pltpu.SemaphoreType.DMA,
      ],
  )


def _prep_body(x_ref, be_ref, o_ref):
  o_ref[...] = x_ref[...] + be_ref[...]


_prep = pl.pallas_call(
    _prep_body,
    out_shape=jax.ShapeDtypeStruct((N, D), jnp.float32),
)


def _node_body(x_ref, p0_ref, p1_ref, eps_ref, w1_ref, b1_ref, g_ref, be_ref,
               w2_ref, b2_ref, go_ref, bo_ref, ben_ref, o_ref):
  h0 = (1.0 + eps_ref[0, 0]) * x_ref[...] + p0_ref[...] + p1_ref[...]
  a = jnp.dot(h0, w1_ref[...], preferred_element_type=jnp.float32) + b1_ref[...]
  m = jnp.mean(a, axis=0, keepdims=True)
  v = jnp.mean((a - m) * (a - m), axis=0, keepdims=True)
  a = jnp.maximum(g_ref[...] * (a - m) * lax.rsqrt(v + 1e-5) + be_ref[...], 0.0)
  a = jnp.dot(a, w2_ref[...], preferred_element_type=jnp.float32) + b2_ref[...]
  m2 = jnp.mean(a, axis=0, keepdims=True)
  v2 = jnp.mean((a - m2) * (a - m2), axis=0, keepdims=True)
  a = jnp.maximum(
      go_ref[...] * (a - m2) * lax.rsqrt(v2 + 1e-5) + bo_ref[...], 0.0)
  o_ref[...] = a + ben_ref[...]


_node = pl.pallas_call(
    _node_body,
    out_shape=jax.ShapeDtypeStruct((N, D), jnp.float32),
)


def _final_body(hp_ref, p0_ref, p1_ref, be_ref, eps_ref, w1_ref, b1_ref,
                g_ref, bei_ref, w2_ref, b2_ref, go_ref, bo_ref, batch_ref,
                wl_ref, bl_ref, o_ref):
  h1 = hp_ref[...] - be_ref[...]
  h0 = (1.0 + eps_ref[0, 0]) * h1 + p0_ref[...] + p1_ref[...]
  a = jnp.dot(h0, w1_ref[...], preferred_element_type=jnp.float32) + b1_ref[...]
  m = jnp.mean(a, axis=0, keepdims=True)
  v = jnp.mean((a - m) * (a - m), axis=0, keepdims=True)
  a = jnp.maximum(g_ref[...] * (a - m) * lax.rsqrt(v + 1e-5) + bei_ref[...],
                  0.0)
  a = jnp.dot(a, w2_ref[...], preferred_element_type=jnp.float32) + b2_ref[...]
  m2 = jnp.mean(a, axis=0, keepdims=True)
  v2 = jnp.mean((a - m2) * (a - m2), axis=0, keepdims=True)
  h2 = jnp.maximum(
      go_ref[...] * (a - m2) * lax.rsqrt(v2 + 1e-5) + bo_ref[...], 0.0)
  gi = lax.broadcasted_iota(jnp.int32, (N, G), 1)
  oh = (batch_ref[...] == gi).astype(jnp.float32)
  pooled = lax.dot_general(oh, h2, (((0,), (0,)), ((), ())),
                           preferred_element_type=jnp.float32)
  o_ref[...] = (
      jnp.dot(pooled, wl_ref[...], preferred_element_type=jnp.float32)
      + bl_ref[...])


_final = pl.pallas_call(
    _final_body,
    out_shape=jax.ShapeDtypeStruct((G, OUT), jnp.float32),
)


def kernel(x, edge_index, batch, edge_attr,
           We0, be0, eps0, m0W1, m0b1, m0g, m0be, m0W2, m0b2, g0, bb0,
           We1, be1, eps1, m1W1, m1b1, m1g, m1be, m1W2, m1b2, g1, bb1,
           Wl, bl):
  src3 = edge_index[0].reshape(NW, NCHUNK, C)
  dst3 = edge_index[1].reshape(NW, NCHUNK, C)
  ea3 = edge_attr.reshape(NW, NCHUNK, C)
  zeros = jnp.zeros((N, D), jnp.float32)
  batchf = batch.reshape(N, 1)
  r2 = lambda a: a.reshape(1, -1)
  s2 = lambda a: a.reshape(1, 1)

  edge = _make_edge_kernel()

  xp0 = _prep(x, r2(be0))
  parts0 = edge(xp0, src3, dst3, ea3, We0.reshape(D), zeros)
  h1p = _node(x, parts0[0], parts0[1], s2(eps0), m0W1, r2(m0b1), r2(m0g),
              r2(m0be), m0W2, r2(m0b2), r2(g0), r2(bb0), r2(be1))
  parts1 = edge(h1p, src3, dst3, ea3, We1.reshape(D), zeros)
  return _final(h1p, parts1[0], parts1[1], r2(be1), s2(eps1), m1W1, r2(m1b1),
                r2(m1g), r2(m1be), m1W2, r2(m1b2), r2(g1), r2(bb1), batchf,
                Wl, bl)


# speedup vs baseline: 8.5279x; 1.5034x over previous
import functools

import jax
import jax.numpy as jnp
from jax import lax
from jax.experimental import pallas as pl
from jax.experimental.pallas import tpu as pltpu
from jax.experimental.pallas import tpu_sc as plsc

N = 10000
E = 320000
D = 128
G = 64
OUT = 10

NC = 2
NS = 16
NW = NC * NS
EW = E // NW
C = 80
NCHUNK = EW // C
NGRP = C // 16
ROWS_PER_TILE = N // NS


def _lane_bcast(v, k):
  idx = jnp.full((16,), k, dtype=jnp.int32)
  dn = lax.GatherDimensionNumbers(
      offset_dims=(), collapsed_slice_dims=(0,), start_index_map=(0,))
  return lax.gather(v, idx[:, None], dn, (1,),
                    mode=lax.GatherScatterMode.PROMISE_IN_BOUNDS)


def _edge_body(xp, src_h, dst_h, ea_h, we_h, zeros_h, out,
               src_v, dst_v, ea_v, r0, r1, r2, r3, r4, we_v, out_acc,
               gsem, ssem):
  cid = lax.axis_index("c")
  sid = lax.axis_index("s")
  wid = sid * NC + cid

  zsl = pl.ds(sid * ROWS_PER_TILE, ROWS_PER_TILE)
  pltpu.sync_copy(zeros_h.at[zsl], out_acc.at[zsl])

  pltpu.sync_copy(src_h.at[wid], src_v)
  pltpu.sync_copy(dst_h.at[wid], dst_v)
  pltpu.sync_copy(ea_h.at[wid], ea_v)
  pltpu.sync_copy(we_h, we_v)

  we_g = [we_v[pl.ds(f * 16, 16)] for f in range(8)]

  plsc.subcore_barrier()

  def compute(c, buf):
    def grp(g, _):
      ea_grp = ea_v[c, pl.ds(g * 16, 16)]
      for k in range(16):
        ea_b = _lane_bcast(ea_grp, k)
        r = g * 16 + k
        for f in range(8):
          sl = pl.ds(f * 16, 16)
          buf[r, sl] = jnp.maximum(buf[r, sl] + ea_b * we_g[f], 0.0)
      return 0
    lax.fori_loop(0, NGRP, grp, 0)

  bufs = [r0, r1]

  def g_issue(c, buf):
    pltpu.async_copy(xp.at[src_v.at[c]], buf, gsem)

  def g_wait(buf):
    pltpu.make_async_copy(xp.at[pl.ds(0, C)], buf, gsem).wait()

  def s_drain():
    pltpu.make_async_copy(
        xp.at[pl.ds(0, C)], out_acc.at[pl.ds(0, C)], ssem).wait()

  g_issue(0, r0)

  def outer(o, _):
    for b in range(2):
      c = o * 2 + b
      buf = bufs[b]
      g_wait(buf)

      @pl.when(c >= 1)
      def _():
        s_drain()

      @pl.when(c <= NCHUNK - 2)
      def _():
        g_issue(c + 1, bufs[(b + 1) % 2])

      compute(c, buf)
      pltpu.async_copy(buf, out_acc.at[dst_v.at[c]], ssem, add=True)
    return 0

  lax.fori_loop(0, (NCHUNK - 1) // 2, outer, 0)
  g_wait(r0)
  s_drain()
  compute(NCHUNK - 1, r0)
  pltpu.sync_copy(r0, out_acc.at[dst_v.at[NCHUNK - 1]], add=True)

  plsc.subcore_barrier()
  pltpu.sync_copy(out_acc.at[zsl], out.at[cid, zsl])


def _make_edge_kernel():
  mesh = plsc.VectorSubcoreMesh(core_axis_name="c", subcore_axis_name="s")
  return pl.kernel(
      _edge_body,
      out_type=jax.ShapeDtypeStruct((NC, N, D), jnp.float32),
      mesh=mesh,
      compiler_params=pltpu.CompilerParams(use_tc_tiling_on_sc=False),
      scratch_types=[
          pltpu.VMEM((NCHUNK, C), jnp.int32),
          pltpu.VMEM((NCHUNK, C), jnp.int32),
          pltpu.VMEM((NCHUNK, C), jnp.float32),
          pltpu.VMEM((C, D), jnp.float32),
          pltpu.VMEM((C, D), jnp.float32),
          pltpu.VMEM((C, D), jnp.float32),
          pltpu.VMEM((C, D), jnp.float32),
          pltpu.VMEM((C, D), jnp.float32),
          pltpu.VMEM((D,), jnp.float32),
          pltpu.VMEM_SHARED((N, D), jnp.float32),
          pltpu.SemaphoreType.DMA,
          pltpu.SemaphoreType.DMA,
      ],
  )


def _prep_body(x_ref, be_ref, o_ref):
  o_ref[...] = x_ref[...] + be_ref[...]


_prep = pl.pallas_call(
    _prep_body,
    out_shape=jax.ShapeDtypeStruct((N, D), jnp.float32),
)


def _node_body(x_ref, p0_ref, p1_ref, eps_ref, w1_ref, b1_ref, g_ref, be_ref,
               w2_ref, b2_ref, go_ref, bo_ref, ben_ref, o_ref):
  h0 = (1.0 + eps_ref[0, 0]) * x_ref[...] + p0_ref[...] + p1_ref[...]
  a = jnp.dot(h0, w1_ref[...], preferred_element_type=jnp.float32) + b1_ref[...]
  m = jnp.mean(a, axis=0, keepdims=True)
  v = jnp.mean((a - m) * (a - m), axis=0, keepdims=True)
  a = jnp.maximum(g_ref[...] * (a - m) * lax.rsqrt(v + 1e-5) + be_ref[...], 0.0)
  a = jnp.dot(a, w2_ref[...], preferred_element_type=jnp.float32) + b2_ref[...]
  m2 = jnp.mean(a, axis=0, keepdims=True)
  v2 = jnp.mean((a - m2) * (a - m2), axis=0, keepdims=True)
  a = jnp.maximum(
      go_ref[...] * (a - m2) * lax.rsqrt(v2 + 1e-5) + bo_ref[...], 0.0)
  o_ref[...] = a + ben_ref[...]


_node = pl.pallas_call(
    _node_body,
    out_shape=jax.ShapeDtypeStruct((N, D), jnp.float32),
)


def _final_body(hp_ref, p0_ref, p1_ref, be_ref, eps_ref, w1_ref, b1_ref,
                g_ref, bei_ref, w2_ref, b2_ref, go_ref, bo_ref, batch_ref,
                wl_ref, bl_ref, o_ref):
  h1 = hp_ref[...] - be_ref[...]
  h0 = (1.0 + eps_ref[0, 0]) * h1 + p0_ref[...] + p1_ref[...]
  a = jnp.dot(h0, w1_ref[...], preferred_element_type=jnp.float32) + b1_ref[...]
  m = jnp.mean(a, axis=0, keepdims=True)
  v = jnp.mean((a - m) * (a - m), axis=0, keepdims=True)
  a = jnp.maximum(g_ref[...] * (a - m) * lax.rsqrt(v + 1e-5) + bei_ref[...],
                  0.0)
  a = jnp.dot(a, w2_ref[...], preferred_element_type=jnp.float32) + b2_ref[...]
  m2 = jnp.mean(a, axis=0, keepdims=True)
  v2 = jnp.mean((a - m2) * (a - m2), axis=0, keepdims=True)
  h2 = jnp.maximum(
      go_ref[...] * (a - m2) * lax.rsqrt(v2 + 1e-5) + bo_ref[...], 0.0)
  gi = lax.broadcasted_iota(jnp.int32, (N, G), 1)
  oh = (batch_ref[...] == gi).astype(jnp.float32)
  pooled = lax.dot_general(oh, h2, (((0,), (0,)), ((), ())),
                           preferred_element_type=jnp.float32)
  o_ref[...] = (
      jnp.dot(pooled, wl_ref[...], preferred_element_type=jnp.float32)
      + bl_ref[...])


_final = pl.pallas_call(
    _final_body,
    out_shape=jax.ShapeDtypeStruct((G, OUT), jnp.float32),
)


def kernel(x, edge_index, batch, edge_attr,
           We0, be0, eps0, m0W1, m0b1, m0g, m0be, m0W2, m0b2, g0, bb0,
           We1, be1, eps1, m1W1, m1b1, m1g, m1be, m1W2, m1b2, g1, bb1,
           Wl, bl):
  src3 = edge_index[0].reshape(NW, NCHUNK, C)
  dst3 = edge_index[1].reshape(NW, NCHUNK, C)
  ea3 = edge_attr.reshape(NW, NCHUNK, C)
  zeros = jnp.zeros((N, D), jnp.float32)
  batchf = batch.reshape(N, 1)
  r2 = lambda a: a.reshape(1, -1)
  s2 = lambda a: a.reshape(1, 1)

  edge = _make_edge_kernel()

  xp0 = _prep(x, r2(be0))
  parts0 = edge(xp0, src3, dst3, ea3, We0.reshape(D), zeros)
  h1p = _node(x, parts0[0], parts0[1], s2(eps0), m0W1, r2(m0b1), r2(m0g),
              r2(m0be), m0W2, r2(m0b2), r2(g0), r2(bb0), r2(be1))
  parts1 = edge(h1p, src3, dst3, ea3, We1.reshape(D), zeros)
  return _final(h1p, parts1[0], parts1[1], r2(be1), s2(eps1), m1W1, r2(m1b1),
                r2(m1g), r2(m1be), m1W2, r2(m1b2), r2(g1), r2(bb1), batchf,
                Wl, bl)


# speedup vs baseline: 8.5928x; 1.0076x over previous
import functools

import jax
import jax.numpy as jnp
from jax import lax
from jax.experimental import pallas as pl
from jax.experimental.pallas import tpu as pltpu
from jax.experimental.pallas import tpu_sc as plsc

N = 10000
E = 320000
D = 128
G = 64
OUT = 10

NC = 2
NS = 16
NW = NC * NS
EW = E // NW
C = 80
NCHUNK = EW // C
NGRP = C // 16
SB = 5
NSUP = NCHUNK // SB
ROWS_PER_TILE = N // NS


def _lane_bcast(v, k):
  idx = jnp.full((16,), k, dtype=jnp.int32)
  dn = lax.GatherDimensionNumbers(
      offset_dims=(), collapsed_slice_dims=(0,), start_index_map=(0,))
  return lax.gather(v, idx[:, None], dn, (1,),
                    mode=lax.GatherScatterMode.PROMISE_IN_BOUNDS)


def _edge_body(xp, src_h, dst_h, ea_h, we_h, zeros_h, out,
               src_vb, dst_vb, ea_vb, r0, r1, s0, s1, we_v, out_acc,
               gsem, ssem, isem):
  cid = lax.axis_index("c")
  sid = lax.axis_index("s")
  wid = sid * NC + cid

  zsl = pl.ds(sid * ROWS_PER_TILE, ROWS_PER_TILE)
  pltpu.sync_copy(zeros_h.at[zsl], out_acc.at[zsl])

  pltpu.sync_copy(src_h.at[wid, 0], src_vb.at[0])
  pltpu.sync_copy(dst_h.at[wid, 0], dst_vb.at[0])
  pltpu.sync_copy(ea_h.at[wid, 0], ea_vb.at[0])
  pltpu.sync_copy(we_h, we_v)

  we_g = [we_v[pl.ds(f * 16, 16)] for f in range(8)]

  def g_issue(c, buf):
    u = c // SB
    pltpu.async_copy(xp.at[src_vb.at[u % 2, c % SB]], buf, gsem)

  def g_wait(buf):
    pltpu.make_async_copy(xp.at[pl.ds(0, C)], buf, gsem).wait()

  def s_drain():
    pltpu.make_async_copy(
        xp.at[pl.ds(0, C)], out_acc.at[pl.ds(0, C)], ssem).wait()

  def i_issue(u):
    slot = u % 2
    pltpu.async_copy(src_h.at[wid, u], src_vb.at[slot], isem)
    pltpu.async_copy(dst_h.at[wid, u], dst_vb.at[slot], isem)
    pltpu.async_copy(ea_h.at[wid, u], ea_vb.at[slot], isem)

  def i_wait():
    def _d(i, _):
      pltpu.make_async_copy(src_h.at[wid, 0], src_vb.at[0], isem).wait()
      return 0
    lax.fori_loop(0, 3, _d, 0)

  g_issue(0, r0)
  plsc.subcore_barrier()

  def compute(c, gbuf, sbuf):
    u = c // SB
    ci = c % SB

    def grp(g, _):
      ea_grp = ea_vb[u % 2, ci, pl.ds(g * 16, 16)]
      for k in range(16):
        ea_b = _lane_bcast(ea_grp, k)
        r = g * 16 + k
        for f in range(8):
          sl = pl.ds(f * 16, 16)
          sbuf[r, sl] = jnp.maximum(gbuf[r, sl] + ea_b * we_g[f], 0.0)
      return 0
    lax.fori_loop(0, NGRP, grp, 0)

  gbufs = [r0, r1]
  sbufs = [s0, s1]

  def outer(o, _):
    for b in range(2):
      c = o * 2 + b
      gbuf = gbufs[b]
      sbuf = sbufs[b]
      g_wait(gbuf)

      @pl.when((c + 1) % SB == 0)
      def _():
        i_wait()

      @pl.when(c <= NCHUNK - 2)
      def _():
        g_issue(c + 1, gbufs[(b + 1) % 2])

      @pl.when(c >= 2)
      def _():
        s_drain()

      @pl.when(jnp.logical_and(c % SB == 1, c // SB <= NSUP - 2))
      def _():
        i_issue(c // SB + 1)

      compute(c, gbuf, sbuf)
      pltpu.async_copy(
          sbuf, out_acc.at[dst_vb.at[(c // SB) % 2, c % SB]], ssem, add=True)
    return 0

  lax.fori_loop(0, (NCHUNK - 1) // 2, outer, 0)
  g_wait(r0)

  def dr(i, _):
    s_drain()
    return 0

  lax.fori_loop(0, 2, dr, 0)
  compute(NCHUNK - 1, r0, s0)
  pltpu.sync_copy(
      s0, out_acc.at[dst_vb.at[(NSUP - 1) % 2, SB - 1]], add=True)

  plsc.subcore_barrier()
  pltpu.sync_copy(out_acc.at[zsl], out.at[cid, zsl])


def _make_edge_kernel():
  mesh = plsc.VectorSubcoreMesh(core_axis_name="c", subcore_axis_name="s")
  return pl.kernel(
      _edge_body,
      out_type=jax.ShapeDtypeStruct((NC, N, D), jnp.float32),
      mesh=mesh,
      compiler_params=pltpu.CompilerParams(use_tc_tiling_on_sc=False),
      scratch_types=[
          pltpu.VMEM((2, SB, C), jnp.int32),
          pltpu.VMEM((2, SB, C), jnp.int32),
          pltpu.VMEM((2, SB, C), jnp.float32),
          pltpu.VMEM((C, D), jnp.float32),
          pltpu.VMEM((C, D), jnp.float32),
          pltpu.VMEM((C, D), jnp.float32),
          pltpu.VMEM((C, D), jnp.float32),
          pltpu.VMEM((D,), jnp.float32),
          pltpu.VMEM_SHARED((N, D), jnp.float32),
          pltpu.SemaphoreType.DMA,
          pltpu.SemaphoreType.DMA,
          pltpu.SemaphoreType.DMA,
      ],
  )


def _prep_body(x_ref, be_ref, o_ref):
  o_ref[...] = x_ref[...] + be_ref[...]


_prep = pl.pallas_call(
    _prep_body,
    out_shape=jax.ShapeDtypeStruct((N, D), jnp.float32),
)


def _node_body(x_ref, p0_ref, p1_ref, eps_ref, w1_ref, b1_ref, g_ref, be_ref,
               w2_ref, b2_ref, go_ref, bo_ref, ben_ref, o_ref):
  h0 = (1.0 + eps_ref[0, 0]) * x_ref[...] + p0_ref[...] + p1_ref[...]
  a = jnp.dot(h0, w1_ref[...], preferred_element_type=jnp.float32) + b1_ref[...]
  m = jnp.mean(a, axis=0, keepdims=True)
  v = jnp.mean((a - m) * (a - m), axis=0, keepdims=True)
  a = jnp.maximum(g_ref[...] * (a - m) * lax.rsqrt(v + 1e-5) + be_ref[...], 0.0)
  a = jnp.dot(a, w2_ref[...], preferred_element_type=jnp.float32) + b2_ref[...]
  m2 = jnp.mean(a, axis=0, keepdims=True)
  v2 = jnp.mean((a - m2) * (a - m2), axis=0, keepdims=True)
  a = jnp.maximum(
      go_ref[...] * (a - m2) * lax.rsqrt(v2 + 1e-5) + bo_ref[...], 0.0)
  o_ref[...] = a + ben_ref[...]


_node = pl.pallas_call(
    _node_body,
    out_shape=jax.ShapeDtypeStruct((N, D), jnp.float32),
)


def _final_body(hp_ref, p0_ref, p1_ref, be_ref, eps_ref, w1_ref, b1_ref,
                g_ref, bei_ref, w2_ref, b2_ref, go_ref, bo_ref, batch_ref,
                wl_ref, bl_ref, o_ref):
  h1 = hp_ref[...] - be_ref[...]
  h0 = (1.0 + eps_ref[0, 0]) * h1 + p0_ref[...] + p1_ref[...]
  a = jnp.dot(h0, w1_ref[...], preferred_element_type=jnp.float32) + b1_ref[...]
  m = jnp.mean(a, axis=0, keepdims=True)
  v = jnp.mean((a - m) * (a - m), axis=0, keepdims=True)
  a = jnp.maximum(g_ref[...] * (a - m) * lax.rsqrt(v + 1e-5) + bei_ref[...],
                  0.0)
  a = jnp.dot(a, w2_ref[...], preferred_element_type=jnp.float32) + b2_ref[...]
  m2 = jnp.mean(a, axis=0, keepdims=True)
  v2 = jnp.mean((a - m2) * (a - m2), axis=0, keepdims=True)
  h2 = jnp.maximum(
      go_ref[...] * (a - m2) * lax.rsqrt(v2 + 1e-5) + bo_ref[...], 0.0)
  gi = lax.broadcasted_iota(jnp.int32, (N, G), 1)
  oh = (batch_ref[...] == gi).astype(jnp.float32)
  pooled = lax.dot_general(oh, h2, (((0,), (0,)), ((), ())),
                           preferred_element_type=jnp.float32)
  o_ref[...] = (
      jnp.dot(pooled, wl_ref[...], preferred_element_type=jnp.float32)
      + bl_ref[...])


_final = pl.pallas_call(
    _final_body,
    out_shape=jax.ShapeDtypeStruct((G, OUT), jnp.float32),
)


def kernel(x, edge_index, batch, edge_attr,
           We0, be0, eps0, m0W1, m0b1, m0g, m0be, m0W2, m0b2, g0, bb0,
           We1, be1, eps1, m1W1, m1b1, m1g, m1be, m1W2, m1b2, g1, bb1,
           Wl, bl):
  src3 = edge_index[0].reshape(NW, NSUP, SB, C)
  dst3 = edge_index[1].reshape(NW, NSUP, SB, C)
  ea3 = edge_attr.reshape(NW, NSUP, SB, C)
  zeros = jnp.zeros((N, D), jnp.float32)
  batchf = batch.reshape(N, 1)
  r2 = lambda a: a.reshape(1, -1)
  s2 = lambda a: a.reshape(1, 1)

  edge = _make_edge_kernel()

  xp0 = _prep(x, r2(be0))
  parts0 = edge(xp0, src3, dst3, ea3, We0.reshape(D), zeros)
  h1p = _node(x, parts0[0], parts0[1], s2(eps0), m0W1, r2(m0b1), r2(m0g),
              r2(m0be), m0W2, r2(m0b2), r2(g0), r2(bb0), r2(be1))
  parts1 = edge(h1p, src3, dst3, ea3, We1.reshape(D), zeros)
  return _final(h1p, parts1[0], parts1[1], r2(be1), s2(eps1), m1W1, r2(m1b1),
                r2(m1g), r2(m1be), m1W2, r2(m1b2), r2(g1), r2(bb1), batchf,
                Wl, bl)


# speedup vs baseline: 8.8753x; 1.0329x over previous
import functools

import jax
import jax.numpy as jnp
from jax import lax
from jax.experimental import pallas as pl
from jax.experimental.pallas import tpu as pltpu
from jax.experimental.pallas import tpu_sc as plsc

N = 10000
E = 320000
D = 128
G = 64
OUT = 10

NC = 2
NS = 16
NW = NC * NS
EW = E // NW
C = 80
NCHUNK = EW // C
NGRP = C // 16
SB = 5
NSUP = NCHUNK // SB
ROWS_PER_TILE = N // NS


def _lane_bcast(v, k):
  idx = jnp.full((16,), k, dtype=jnp.int32)
  dn = lax.GatherDimensionNumbers(
      offset_dims=(), collapsed_slice_dims=(0,), start_index_map=(0,))
  return lax.gather(v, idx[:, None], dn, (1,),
                    mode=lax.GatherScatterMode.PROMISE_IN_BOUNDS)


def _edge_body(xp, src_h, dst_h, ea_h, we_h, be_h, zeros_h, out,
               src_vb, dst_vb, ea_vb, r0, r1, r2, r3, we_v, be_v, out_acc,
               gsem, ssem, isem):
  cid = lax.axis_index("c")
  sid = lax.axis_index("s")
  wid = sid * NC + cid

  zsl = pl.ds(sid * ROWS_PER_TILE, ROWS_PER_TILE)
  pltpu.sync_copy(zeros_h.at[zsl], out_acc.at[zsl])

  pltpu.sync_copy(src_h.at[wid, 0], src_vb.at[0])
  pltpu.sync_copy(dst_h.at[wid, 0], dst_vb.at[0])
  pltpu.sync_copy(ea_h.at[wid, 0], ea_vb.at[0])
  pltpu.sync_copy(we_h, we_v)
  pltpu.sync_copy(be_h, be_v)

  we_g = [we_v[pl.ds(f * 16, 16)] for f in range(8)]
  be_g = [be_v[pl.ds(f * 16, 16)] for f in range(8)]

  bufs = [r0, r1, r2, r3]

  def g_issue(c, buf):
    u = c // SB
    pltpu.async_copy(xp.at[src_vb.at[u % 2, c % SB]], buf, gsem)

  def g_wait(buf):
    pltpu.make_async_copy(xp.at[pl.ds(0, C)], buf, gsem).wait()

  def s_drain():
    pltpu.make_async_copy(
        xp.at[pl.ds(0, C)], out_acc.at[pl.ds(0, C)], ssem).wait()

  def i_issue(u):
    slot = u % 2
    pltpu.async_copy(src_h.at[wid, u], src_vb.at[slot], isem)
    pltpu.async_copy(dst_h.at[wid, u], dst_vb.at[slot], isem)
    pltpu.async_copy(ea_h.at[wid, u], ea_vb.at[slot], isem)

  def i_wait():
    def _d(i, _):
      pltpu.make_async_copy(src_h.at[wid, 0], src_vb.at[0], isem).wait()
      return 0
    lax.fori_loop(0, 3, _d, 0)

  g_issue(0, r0)
  g_issue(1, r1)
  plsc.subcore_barrier()

  def compute(c, buf):
    u = c // SB
    ci = c % SB

    def grp(g, _):
      ea_grp = ea_vb[u % 2, ci, pl.ds(g * 16, 16)]
      for k in range(16):
        ea_b = _lane_bcast(ea_grp, k)
        r = g * 16 + k
        for f in range(8):
          sl = pl.ds(f * 16, 16)
          buf[r, sl] = jnp.maximum(
              buf[r, sl] + (ea_b * we_g[f] + be_g[f]), 0.0)
      return 0
    lax.fori_loop(0, NGRP, grp, 0)

  def s_issue(c, buf):
    pltpu.async_copy(
        buf, out_acc.at[dst_vb.at[(c // SB) % 2, c % SB]], ssem, add=True)

  def outer(o, _):
    for b in range(4):
      c = o * 4 + b
      buf = bufs[b]
      g_wait(buf)

      @pl.when(jnp.logical_and((c + 2) % SB == 0, c <= NCHUNK - 3))
      def _():
        i_wait()

      @pl.when(c >= 2)
      def _():
        s_drain()

      @pl.when(c <= NCHUNK - 3)
      def _():
        g_issue(c + 2, bufs[(b + 2) % 4])

      @pl.when(jnp.logical_and(c % SB == 1, c // SB <= NSUP - 2))
      def _():
        i_issue(c // SB + 1)

      compute(c, buf)
      s_issue(c, buf)
    return 0

  lax.fori_loop(0, (NCHUNK - 1) // 4, outer, 0)
  g_wait(r0)

  def dr(i, _):
    s_drain()
    return 0

  lax.fori_loop(0, 2, dr, 0)
  compute(NCHUNK - 1, r0)
  pltpu.sync_copy(
      r0, out_acc.at[dst_vb.at[(NSUP - 1) % 2, SB - 1]], add=True)

  plsc.subcore_barrier()
  pltpu.sync_copy(out_acc.at[zsl], out.at[cid, zsl])


def _make_edge_kernel():
  mesh = plsc.VectorSubcoreMesh(core_axis_name="c", subcore_axis_name="s")
  return pl.kernel(
      _edge_body,
      out_type=jax.ShapeDtypeStruct((NC, N, D), jnp.float32),
      mesh=mesh,
      compiler_params=pltpu.CompilerParams(use_tc_tiling_on_sc=False),
      scratch_types=[
          pltpu.VMEM((2, SB, C), jnp.int32),
          pltpu.VMEM((2, SB, C), jnp.int32),
          pltpu.VMEM((2, SB, C), jnp.float32),
          pltpu.VMEM((C, D), jnp.float32),
          pltpu.VMEM((C, D), jnp.float32),
          pltpu.VMEM((C, D), jnp.float32),
          pltpu.VMEM((C, D), jnp.float32),
          pltpu.VMEM((D,), jnp.float32),
          pltpu.VMEM((D,), jnp.float32),
          pltpu.VMEM_SHARED((N, D), jnp.float32),
          pltpu.SemaphoreType.DMA,
          pltpu.SemaphoreType.DMA,
          pltpu.SemaphoreType.DMA,
      ],
  )


def _node_body(x_ref, p0_ref, p1_ref, eps_ref, w1_ref, b1_ref, g_ref, be_ref,
               w2_ref, b2_ref, go_ref, bo_ref, o_ref):
  h0 = (1.0 + eps_ref[0, 0]) * x_ref[...] + p0_ref[...] + p1_ref[...]
  a = jnp.dot(h0, w1_ref[...], preferred_element_type=jnp.float32) + b1_ref[...]
  m = jnp.mean(a, axis=0, keepdims=True)
  v = jnp.mean((a - m) * (a - m), axis=0, keepdims=True)
  a = jnp.maximum(g_ref[...] * (a - m) * lax.rsqrt(v + 1e-5) + be_ref[...], 0.0)
  a = jnp.dot(a, w2_ref[...], preferred_element_type=jnp.float32) + b2_ref[...]
  m2 = jnp.mean(a, axis=0, keepdims=True)
  v2 = jnp.mean((a - m2) * (a - m2), axis=0, keepdims=True)
  o_ref[...] = jnp.maximum(
      go_ref[...] * (a - m2) * lax.rsqrt(v2 + 1e-5) + bo_ref[...], 0.0)


_node = pl.pallas_call(
    _node_body,
    out_shape=jax.ShapeDtypeStruct((N, D), jnp.float32),
)


def _final_body(h1_ref, p0_ref, p1_ref, eps_ref, w1_ref, b1_ref,
                g_ref, bei_ref, w2_ref, b2_ref, go_ref, bo_ref, batch_ref,
                wl_ref, bl_ref, o_ref):
  h0 = (1.0 + eps_ref[0, 0]) * h1_ref[...] + p0_ref[...] + p1_ref[...]
  a = jnp.dot(h0, w1_ref[...], preferred_element_type=jnp.float32) + b1_ref[...]
  m = jnp.mean(a, axis=0, keepdims=True)
  v = jnp.mean((a - m) * (a - m), axis=0, keepdims=True)
  a = jnp.maximum(g_ref[...] * (a - m) * lax.rsqrt(v + 1e-5) + bei_ref[...],
                  0.0)
  a = jnp.dot(a, w2_ref[...], preferred_element_type=jnp.float32) + b2_ref[...]
  m2 = jnp.mean(a, axis=0, keepdims=True)
  v2 = jnp.mean((a - m2) * (a - m2), axis=0, keepdims=True)
  h2 = jnp.maximum(
      go_ref[...] * (a - m2) * lax.rsqrt(v2 + 1e-5) + bo_ref[...], 0.0)
  gi = lax.broadcasted_iota(jnp.int32, (N, G), 1)
  oh = (batch_ref[...] == gi).astype(jnp.float32)
  pooled = lax.dot_general(oh, h2, (((0,), (0,)), ((), ())),
                           preferred_element_type=jnp.float32)
  o_ref[...] = (
      jnp.dot(pooled, wl_ref[...], preferred_element_type=jnp.float32)
      + bl_ref[...])


_final = pl.pallas_call(
    _final_body,
    out_shape=jax.ShapeDtypeStruct((G, OUT), jnp.float32),
)


def kernel(x, edge_index, batch, edge_attr,
           We0, be0, eps0, m0W1, m0b1, m0g, m0be, m0W2, m0b2, g0, bb0,
           We1, be1, eps1, m1W1, m1b1, m1g, m1be, m1W2, m1b2, g1, bb1,
           Wl, bl):
  src3 = edge_index[0].reshape(NW, NSUP, SB, C)
  dst3 = edge_index[1].reshape(NW, NSUP, SB, C)
  ea3 = edge_attr.reshape(NW, NSUP, SB, C)
  zeros = jnp.zeros((N, D), jnp.float32)
  batchf = batch.reshape(N, 1)
  r2 = lambda a: a.reshape(1, -1)
  s2 = lambda a: a.reshape(1, 1)

  edge = _make_edge_kernel()

  parts0 = edge(x, src3, dst3, ea3, We0.reshape(D), be0, zeros)
  h1 = _node(x, parts0[0], parts0[1], s2(eps0), m0W1, r2(m0b1), r2(m0g),
             r2(m0be), m0W2, r2(m0b2), r2(g0), r2(bb0))
  parts1 = edge(h1, src3, dst3, ea3, We1.reshape(D), be1, zeros)
  return _final(h1, parts1[0], parts1[1], s2(eps1), m1W1, r2(m1b1),
                r2(m1g), r2(m1be), m1W2, r2(m1b2), r2(g1), r2(bb1), batchf,
                Wl, bl)


# speedup vs baseline: 9.1894x; 1.0354x over previous
import functools

import jax
import jax.numpy as jnp
from jax import lax
from jax.experimental import pallas as pl
from jax.experimental.pallas import tpu as pltpu
from jax.experimental.pallas import tpu_sc as plsc

N = 10000
E = 320000
D = 128
G = 64
OUT = 10

NC = 2
NS = 16
NW = NC * NS
EW = E // NW
C = 80
NCHUNK = EW // C
NGRP = C // 16
SB = 5
NSUP = NCHUNK // SB
ROWS_PER_TILE = N // NS


def _lane_bcast(v, k):
  idx = jnp.full((16,), k, dtype=jnp.int32)
  dn = lax.GatherDimensionNumbers(
      offset_dims=(), collapsed_slice_dims=(0,), start_index_map=(0,))
  return lax.gather(v, idx[:, None], dn, (1,),
                    mode=lax.GatherScatterMode.PROMISE_IN_BOUNDS)


def _edge_body(xp, src_h, dst_h, ea_h, we_h, be_h, zeros_h, out,
               src_vb, dst_vb, ea_vb, r0, r1, r2, r3, we_v, be_v, out_acc,
               gsem, ssem, isem):
  cid = lax.axis_index("c")
  sid = lax.axis_index("s")
  wid = sid * NC + cid

  zsl = pl.ds(sid * ROWS_PER_TILE, ROWS_PER_TILE)
  pltpu.sync_copy(zeros_h.at[zsl], out_acc.at[zsl])

  pltpu.sync_copy(src_h.at[wid, 0], src_vb.at[0])
  pltpu.sync_copy(dst_h.at[wid, 0], dst_vb.at[0])
  pltpu.sync_copy(ea_h.at[wid, 0], ea_vb.at[0])
  pltpu.sync_copy(we_h, we_v)
  pltpu.sync_copy(be_h, be_v)

  we_g = [we_v[pl.ds(f * 16, 16)] for f in range(8)]
  be_g = [be_v[pl.ds(f * 16, 16)] for f in range(8)]

  bufs = [r0, r1, r2, r3]

  def g_issue(c, buf):
    u = c // SB
    pltpu.async_copy(xp.at[src_vb.at[u % 2, c % SB]], buf, gsem)

  def g_wait(buf):
    pltpu.make_async_copy(xp.at[pl.ds(0, C)], buf, gsem).wait()

  def s_drain():
    pltpu.make_async_copy(
        xp.at[pl.ds(0, C)], out_acc.at[pl.ds(0, C)], ssem).wait()

  def i_issue(u):
    slot = u % 2
    pltpu.async_copy(src_h.at[wid, u], src_vb.at[slot], isem)
    pltpu.async_copy(dst_h.at[wid, u], dst_vb.at[slot], isem)
    pltpu.async_copy(ea_h.at[wid, u], ea_vb.at[slot], isem)

  def i_wait():
    def _d(i, _):
      pltpu.make_async_copy(src_h.at[wid, 0], src_vb.at[0], isem).wait()
      return 0
    lax.fori_loop(0, 3, _d, 0)

  g_issue(0, r0)
  g_issue(1, r1)
  plsc.subcore_barrier()

  def compute(c, buf):
    u = c // SB
    ci = c % SB

    def grp(g, _):
      ea_grp = ea_vb[u % 2, ci, pl.ds(g * 16, 16)]
      for k in range(16):
        ea_b = _lane_bcast(ea_grp, k)
        r = g * 16 + k
        for f in range(8):
          sl = pl.ds(f * 16, 16)
          buf[r, sl] = jnp.maximum(
              buf[r, sl] + (ea_b * we_g[f] + be_g[f]), 0.0)
      return 0
    lax.fori_loop(0, NGRP, grp, 0)

  def s_issue(c, buf):
    pltpu.async_copy(
        buf, out_acc.at[dst_vb.at[(c // SB) % 2, c % SB]], ssem, add=True)

  def outer(o, _):
    for b in range(4):
      c = o * 4 + b
      buf = bufs[b]
      g_wait(buf)

      @pl.when(jnp.logical_and((c + 2) % SB == 0, c <= NCHUNK - 3))
      def _():
        i_wait()

      @pl.when(c >= 2)
      def _():
        s_drain()

      @pl.when(c <= NCHUNK - 3)
      def _():
        g_issue(c + 2, bufs[(b + 2) % 4])

      @pl.when(jnp.logical_and(c % SB == 1, c // SB <= NSUP - 2))
      def _():
        i_issue(c // SB + 1)

      compute(c, buf)
      s_issue(c, buf)
    return 0

  lax.fori_loop(0, (NCHUNK - 1) // 4, outer, 0)
  g_wait(r0)

  def dr(i, _):
    s_drain()
    return 0

  lax.fori_loop(0, 2, dr, 0)
  compute(NCHUNK - 1, r0)
  pltpu.sync_copy(
      r0, out_acc.at[dst_vb.at[(NSUP - 1) % 2, SB - 1]], add=True)

  plsc.subcore_barrier()
  pltpu.sync_copy(out_acc.at[zsl], out.at[cid, zsl])


def _make_edge_kernel():
  mesh = plsc.VectorSubcoreMesh(core_axis_name="c", subcore_axis_name="s")
  return pl.kernel(
      _edge_body,
      out_type=jax.ShapeDtypeStruct((NC, N, D), jnp.float32),
      mesh=mesh,
      compiler_params=pltpu.CompilerParams(use_tc_tiling_on_sc=False),
      scratch_types=[
          pltpu.VMEM((2, SB, C), jnp.int32),
          pltpu.VMEM((2, SB, C), jnp.int32),
          pltpu.VMEM((2, SB, C), jnp.float32),
          pltpu.VMEM((C, D), jnp.float32),
          pltpu.VMEM((C, D), jnp.float32),
          pltpu.VMEM((C, D), jnp.float32),
          pltpu.VMEM((C, D), jnp.float32),
          pltpu.VMEM((D,), jnp.float32),
          pltpu.VMEM((D,), jnp.float32),
          pltpu.VMEM_SHARED((N, D), jnp.float32),
          pltpu.SemaphoreType.DMA,
          pltpu.SemaphoreType.DMA,
          pltpu.SemaphoreType.DMA,
      ],
  )


def _node_body(x_ref, p_ref, eps_ref, w1_ref, b1_ref, g_ref, be_ref,
               w2_ref, b2_ref, go_ref, bo_ref, o_ref):
  h0 = (1.0 + eps_ref[0, 0]) * x_ref[...] + p_ref[0] + p_ref[1]
  a = jnp.dot(h0, w1_ref[...], preferred_element_type=jnp.float32) + b1_ref[...]
  m = jnp.mean(a, axis=0, keepdims=True)
  v = jnp.mean((a - m) * (a - m), axis=0, keepdims=True)
  a = jnp.maximum(g_ref[...] * (a - m) * lax.rsqrt(v + 1e-5) + be_ref[...], 0.0)
  a = jnp.dot(a, w2_ref[...], preferred_element_type=jnp.float32) + b2_ref[...]
  m2 = jnp.mean(a, axis=0, keepdims=True)
  v2 = jnp.mean((a - m2) * (a - m2), axis=0, keepdims=True)
  o_ref[...] = jnp.maximum(
      go_ref[...] * (a - m2) * lax.rsqrt(v2 + 1e-5) + bo_ref[...], 0.0)


_node = pl.pallas_call(
    _node_body,
    out_shape=jax.ShapeDtypeStruct((N, D), jnp.float32),
)


def _final_body(h1_ref, p_ref, eps_ref, w1_ref, b1_ref,
                g_ref, bei_ref, w2_ref, b2_ref, go_ref, bo_ref, batch_ref,
                wl_ref, bl_ref, o_ref):
  h0 = (1.0 + eps_ref[0, 0]) * h1_ref[...] + p_ref[0] + p_ref[1]
  a = jnp.dot(h0, w1_ref[...], preferred_element_type=jnp.float32) + b1_ref[...]
  m = jnp.mean(a, axis=0, keepdims=True)
  v = jnp.mean((a - m) * (a - m), axis=0, keepdims=True)
  a = jnp.maximum(g_ref[...] * (a - m) * lax.rsqrt(v + 1e-5) + bei_ref[...],
                  0.0)
  a = jnp.dot(a, w2_ref[...], preferred_element_type=jnp.float32) + b2_ref[...]
  m2 = jnp.mean(a, axis=0, keepdims=True)
  v2 = jnp.mean((a - m2) * (a - m2), axis=0, keepdims=True)
  h2 = jnp.maximum(
      go_ref[...] * (a - m2) * lax.rsqrt(v2 + 1e-5) + bo_ref[...], 0.0)
  gi = lax.broadcasted_iota(jnp.int32, (N, G), 1)
  oh = (batch_ref[...] == gi).astype(jnp.float32)
  pooled = lax.dot_general(oh, h2, (((0,), (0,)), ((), ())),
                           preferred_element_type=jnp.float32)
  o_ref[...] = (
      jnp.dot(pooled, wl_ref[...], preferred_element_type=jnp.float32)
      + bl_ref[...])


_final = pl.pallas_call(
    _final_body,
    out_shape=jax.ShapeDtypeStruct((G, OUT), jnp.float32),
)


def kernel(x, edge_index, batch, edge_attr,
           We0, be0, eps0, m0W1, m0b1, m0g, m0be, m0W2, m0b2, g0, bb0,
           We1, be1, eps1, m1W1, m1b1, m1g, m1be, m1W2, m1b2, g1, bb1,
           Wl, bl):
  src3 = edge_index[0].reshape(NW, NSUP, SB, C)
  dst3 = edge_index[1].reshape(NW, NSUP, SB, C)
  ea3 = edge_attr.reshape(NW, NSUP, SB, C)
  zeros = jnp.zeros((N, D), jnp.float32)
  batchf = batch.reshape(N, 1)
  r2 = lambda a: a.reshape(1, -1)
  s2 = lambda a: a.reshape(1, 1)

  edge = _make_edge_kernel()

  parts0 = edge(x, src3, dst3, ea3, We0.reshape(D), be0, zeros)
  h1 = _node(x, parts0, s2(eps0), m0W1, r2(m0b1), r2(m0g),
             r2(m0be), m0W2, r2(m0b2), r2(g0), r2(bb0))
  parts1 = edge(h1, src3, dst3, ea3, We1.reshape(D), be1, zeros)
  return _final(h1, parts1, s2(eps1), m1W1, r2(m1b1),
                r2(m1g), r2(m1be), m1W2, r2(m1b2), r2(g1), r2(bb1), batchf,
                Wl, bl)
